# baseline (speedup 1.0000x reference)
import dataclasses
import functools

import jax
import jax.numpy as jnp
from jax import lax
from jax.experimental import pallas as pl
from jax.experimental.pallas import tpu as pltpu
from jax.experimental.pallas import tpu_sc as plsc

N_NODES = 10000
N_EDGES = 320000
N_GRAPHS = 64
N_CLASSES = 10

_NC = 2
_NS = 16
_NW = _NC * _NS
_K = 64
_CHUNKS_PER_W = 160
_N_CHUNKS = _CHUNKS_PER_W * _NW
_E_PAD = _N_CHUNKS * _K

_ROWS_PER_TILE = 624
_ROWS_REM = N_NODES - _NS * _ROWS_PER_TILE

_HIGH = lax.Precision.HIGHEST


def _dot(a, b):
    return lax.dot_general(a, b, (((1,), (0,)), ((), ())),
                           preferred_element_type=jnp.float32,
                           precision=_HIGH)


def _mm(x, w):
    def body(x_ref, w_ref, o_ref):
        o_ref[...] = _dot(x_ref[...], w_ref[...])
    return pl.pallas_call(
        body,
        out_shape=jax.ShapeDtypeStruct((x.shape[0], w.shape[1]), jnp.float32),
    )(x, w)


def _fuse(acc, b, w):
    def body(a_ref, b_ref, w_ref, o_ref):
        h = jnp.maximum(a_ref[0] + a_ref[1] + b_ref[...], 0.0)
        o_ref[...] = _dot(h, w_ref[...])
    return pl.pallas_call(
        body,
        out_shape=jax.ShapeDtypeStruct((acc.shape[1], w.shape[1]), jnp.float32),
    )(acc, b.reshape(1, -1), w)


def _final(acc, b, batch2d, wlin, blin):
    def body(a_ref, b_ref, bt_ref, wl_ref, bl_ref, o_ref):
        out3 = a_ref[0] + a_ref[1] + b_ref[...]
        gi = lax.broadcasted_iota(jnp.int32, (N_NODES, N_GRAPHS), 1)
        onehot = (bt_ref[...] == gi).astype(jnp.float32)
        sums = lax.dot_general(onehot, out3, (((0,), (0,)), ((), ())),
                               preferred_element_type=jnp.float32,
                               precision=_HIGH)
        ones = jnp.ones((N_NODES, 1), jnp.float32)
        counts = lax.dot_general(onehot, ones, (((0,), (0,)), ((), ())),
                                 preferred_element_type=jnp.float32,
                                 precision=_HIGH)
        pooled = sums / jnp.maximum(counts, 1.0)
        o_ref[...] = _dot(pooled, wl_ref[...]) + bl_ref[...]
    return pl.pallas_call(
        body,
        out_shape=jax.ShapeDtypeStruct((N_GRAPHS, N_CLASSES), jnp.float32),
    )(acc, b.reshape(1, -1), batch2d, wlin, blin.reshape(1, -1))


def _make_scatter(d):
    mesh = plsc.VectorSubcoreMesh(core_axis_name="c", subcore_axis_name="s")
    cp = pltpu.CompilerParams()
    if "needs_layout_passes" in pltpu.CompilerParams.__dataclass_fields__:
        cp = dataclasses.replace(cp, needs_layout_passes=False)
    if d < 128 and "use_tc_tiling_on_sc" in pltpu.CompilerParams.__dataclass_fields__:
        cp = dataclasses.replace(cp, use_tc_tiling_on_sc=False)

    nj = d // 16

    @functools.partial(
        pl.kernel,
        compiler_params=cp,
        out_type=jax.ShapeDtypeStruct((_NC, N_NODES, d), jnp.float32),
        mesh=mesh,
        scratch_types=(
            [pltpu.VMEM((_K, d), jnp.float32) for _ in range(4)]
            + [pltpu.VMEM((2 * _K,), jnp.int32) for _ in range(8)]
            + [pltpu.VMEM((_K,), jnp.int32) for _ in range(8)]
            + [pltpu.VMEM_SHARED((N_NODES, d), jnp.float32)]
            + [pltpu.SemaphoreType.DMA for _ in range(16)]
        ),
    )
    def sc_kernel(ed_hbm, dst_hbm, h_hbm, z_hbm, out_hbm, *scr):
        rows = scr[0:4]
        ed = scr[4:12]
        dv = scr[12:20]
        acc = scr[20]
        gsem = scr[21:25]
        ssem = scr[25:29]
        esem = scr[29:37]

        c = lax.axis_index("c")
        s = lax.axis_index("s")
        wid = s * _NC + c
        r0 = s * _ROWS_PER_TILE

        def start_edata(i, se):
            ch = i * _NW + wid
            pltpu.async_copy(ed_hbm.at[pl.ds(ch * 2 * _K, 2 * _K)],
                             ed[se], esem[se])
            pltpu.async_copy(dst_hbm.at[pl.ds(ch * _K, _K)], dv[se], esem[se])

        def wait_edata(i, se):
            ch = i * _NW + wid
            pltpu.make_async_copy(ed_hbm.at[pl.ds(ch * 2 * _K, 2 * _K)],
                                  ed[se], esem[se]).wait()
            pltpu.make_async_copy(dst_hbm.at[pl.ds(ch * _K, _K)],
                                  dv[se], esem[se]).wait()

        def start_gather(sr, se):
            pltpu.async_copy(h_hbm.at[ed[se].at[pl.ds(0, _K)]],
                             rows[sr], gsem[sr])

        def wait_gather(sr, se):
            pltpu.make_async_copy(h_hbm.at[ed[se].at[pl.ds(0, _K)]],
                                  rows[sr], gsem[sr]).wait()

        def start_scatter(sr, se):
            pltpu.async_copy(rows[sr], acc.at[dv[se]], ssem[sr], add=True)

        def wait_scatter(sr, se):
            pltpu.make_async_copy(rows[sr], acc.at[dv[se]], ssem[sr]).wait()

        def multiply(sr, se):
            @pl.loop(0, _K, step=4)
            def _(k0):
                for kk in range(4):
                    k = k0 + kk
                    wb = plsc.bitcast(
                        plsc.load_gather(ed[se], [jnp.full((16,), _K, jnp.int32) + k]),
                        jnp.float32)
                    for j in range(nj):
                        sl = (k, pl.ds(j * 16, 16))
                        rows[sr][sl] = rows[sr][sl] * wb

        pltpu.sync_copy(z_hbm.at[pl.ds(r0, _ROWS_PER_TILE)],
                        acc.at[pl.ds(r0, _ROWS_PER_TILE)])

        @pl.when(s == _NS - 1)
        def _():
            pltpu.sync_copy(z_hbm.at[pl.ds(_NS * _ROWS_PER_TILE, _ROWS_REM)],
                            acc.at[pl.ds(_NS * _ROWS_PER_TILE, _ROWS_REM)])

        plsc.subcore_barrier()

        for p in range(6):
            start_edata(p, p)
        for p in range(2):
            wait_edata(p, p)
            start_gather(p, p)

        @pl.loop(0, _CHUNKS_PER_W, step=8)
        def _(i0):
            for b in range(8):
                i = i0 + b
                sr, se = b % 4, b
                wait_gather(sr, se)
                multiply(sr, se)
                start_scatter(sr, se)

                @pl.when(i >= 2)
                def _():
                    wait_scatter((b + 2) % 4, (b + 6) % 8)

                @pl.when(i + 2 < _CHUNKS_PER_W)
                def _():
                    wait_edata(i + 2, (b + 2) % 8)
                    start_gather((b + 2) % 4, (b + 2) % 8)

                @pl.when(i + 6 < _CHUNKS_PER_W)
                def _():
                    start_edata(i + 6, (b + 6) % 8)

        wait_scatter(2, 6)
        wait_scatter(3, 7)

        plsc.subcore_barrier()
        pltpu.sync_copy(acc.at[pl.ds(r0, _ROWS_PER_TILE)],
                        out_hbm.at[c, pl.ds(r0, _ROWS_PER_TILE)])

        @pl.when(s == _NS - 1)
        def _():
            pltpu.sync_copy(acc.at[pl.ds(_NS * _ROWS_PER_TILE, _ROWS_REM)],
                            out_hbm.at[c, pl.ds(_NS * _ROWS_PER_TILE, _ROWS_REM)])

    return sc_kernel


_scatter128 = _make_scatter(128)
_scatter64 = _make_scatter(64)


@jax.jit
def kernel(x, edge_index, batch, edge_weights, W1, b1, W2, b2, W3, b3,
           Wlin, blin):
    src = edge_index[0].astype(jnp.int32)
    dst = edge_index[1].astype(jnp.int32)
    pad = _E_PAD - N_EDGES
    src_p = jnp.concatenate([src, jnp.zeros((pad,), jnp.int32)])
    dst_p = jnp.concatenate([dst, jnp.zeros((pad,), jnp.int32)])
    w_p = jnp.concatenate([edge_weights.astype(jnp.float32),
                           jnp.zeros((pad,), jnp.float32)])
    wbits = lax.bitcast_convert_type(w_p, jnp.int32)
    ed = jnp.concatenate([src_p.reshape(_N_CHUNKS, 1, _K),
                          wbits.reshape(_N_CHUNKS, 1, _K)], axis=1).reshape(-1)

    z128 = jnp.zeros((N_NODES, 128), jnp.float32)
    z64 = jnp.zeros((N_NODES, 64), jnp.float32)
    batch2d = batch.astype(jnp.int32).reshape(N_NODES, 1)

    h1 = _mm(x, W1)
    a1 = _scatter128(ed, dst_p, h1, z128)
    h2 = _fuse(a1, b1, W2)
    a2 = _scatter128(ed, dst_p, h2, z128)
    h3 = _fuse(a2, b2, W3)
    a3 = _scatter64(ed, dst_p, h3, z64)
    return _final(a3, b3, batch2d, Wlin, blin)
